# baseline (speedup 1.0000x reference)
import functools

import jax
import jax.numpy as jnp
from jax import lax
from jax.experimental import pallas as pl
from jax.experimental.pallas import tpu as pltpu
from jax.experimental.pallas import tpu_sc as plsc

_B = 128
_D = 32768
_L = 16
_GRP = 8
_NGRP = _D // (_L * _GRP)
_N_BISECT = 22
_WORKERS = 32
_ROWS_PER_W = _B // _WORKERS


def _process_row(row_v, cand_v, gmax_v):
    @plsc.parallel_loop(0, _NGRP, unroll=8, carry=row_v[pl.ds(0, _L)])
    def acc(g, acc):
        base = g * (_L * _GRP)
        c0 = jnp.maximum(row_v[pl.ds(base, _L)], row_v[pl.ds(base + _L, _L)])
        c1 = jnp.maximum(row_v[pl.ds(base + 2 * _L, _L)],
                         row_v[pl.ds(base + 3 * _L, _L)])
        c2 = jnp.maximum(row_v[pl.ds(base + 4 * _L, _L)],
                         row_v[pl.ds(base + 5 * _L, _L)])
        c3 = jnp.maximum(row_v[pl.ds(base + 6 * _L, _L)],
                         row_v[pl.ds(base + 7 * _L, _L)])
        gv = jnp.maximum(jnp.maximum(c0, c1), jnp.maximum(c2, c3))
        gmax_v[pl.ds(g * _L, _L)] = gv
        return jnp.maximum(acc, gv)

    m = jnp.max(acc)
    thr = m - 1.0

    @plsc.parallel_loop(0, _NGRP, unroll=2, carry=jnp.int32(0))
    def off(g, off):
        anyg = jnp.any(gmax_v[pl.ds(g * _L, _L)] > thr)

        def slow(off):
            base = g * (_L * _GRP)
            for c in range(_GRP):
                v = row_v[pl.ds(base + c * _L, _L)]
                mk = v > thr
                mi = jnp.where(mk, 1, 0).astype(jnp.int32)
                cs = plsc.cumsum(mi)
                pos = cs - 1 + off
                plsc.store_scatter(cand_v, [pos], v, mask=mk)
                off = off + jnp.sum(mi)
            return off

        return lax.cond(anyg, slow, lambda off: off, off)

    pad = jnp.full((_L,), 0.0, jnp.float32) + thr
    cand_v[pl.ds(off, _L)] = pad
    cand_v[pl.ds(off + _L, _L)] = pad
    nch = off // _L + 1

    def probe(t):
        def pb(i, a):
            return a + jnp.maximum(cand_v[pl.ds(i * _L, _L)] - t, 0.0)

        return jnp.sum(lax.fori_loop(0, nch, pb, jnp.zeros((_L,), jnp.float32)))

    def bis_body(_, carry):
        lo, hi = carry
        mid = 0.5 * (lo + hi)
        ge = probe(mid) >= 1.0
        lo = jnp.where(ge, mid, lo)
        hi = jnp.where(ge, hi, mid)
        return lo, hi

    lo, _hi = lax.fori_loop(0, _N_BISECT, bis_body, (thr, m))

    def ref_body(i, carry):
        ka, sa = carry
        v = cand_v[pl.ds(i * _L, _L)]
        mk = v > lo
        return (ka + jnp.where(mk, 1.0, 0.0), sa + jnp.where(mk, v, 0.0))

    ka, sa = lax.fori_loop(
        0, nch, ref_body,
        (jnp.zeros((_L,), jnp.float32), jnp.zeros((_L,), jnp.float32)))
    sa_b = jnp.zeros((_L,), jnp.float32) + jnp.sum(sa)
    ka_b = jnp.zeros((_L,), jnp.float32) + jnp.sum(ka)
    tau = (sa_b - 1.0) / ka_b

    @plsc.parallel_loop(0, _NGRP, unroll=8)
    def _(g):
        base = g * (_L * _GRP)
        for c in range(_GRP):
            idx = pl.ds(base + c * _L, _L)
            row_v[idx] = jnp.maximum(row_v[idx] - tau, 0.0)


def _sc_body(x_hbm, out_hbm, row_a, row_b, cand_v, gmax_v,
             in_a, in_b, out_a, out_b):
    wid = lax.axis_index("s") * 2 + lax.axis_index("c")
    base = wid * _ROWS_PER_W

    bufs = [(row_a, in_a, out_a), (row_b, in_b, out_b)]
    in_h = [None, None]
    out_h = [None, None]
    in_h[0] = pltpu.async_copy(x_hbm.at[base], row_a, in_a)
    for j in range(_ROWS_PER_W):
        buf, insem, outsem = bufs[j % 2]
        in_h[j % 2].wait()
        if j + 1 < _ROWS_PER_W:
            nbuf, ninsem, _ = bufs[(j + 1) % 2]
            if j >= 1:
                out_h[(j + 1) % 2].wait()
            in_h[(j + 1) % 2] = pltpu.async_copy(
                x_hbm.at[base + j + 1], nbuf, ninsem)
        _process_row(buf, cand_v, gmax_v)
        out_h[j % 2] = pltpu.async_copy(buf, out_hbm.at[base + j], outsem)
    out_h[(_ROWS_PER_W - 1) % 2].wait()
    out_h[(_ROWS_PER_W - 2) % 2].wait()


def kernel(x):
    return pl.kernel(
        _sc_body,
        out_type=jax.ShapeDtypeStruct((_B, _D), jnp.float32),
        mesh=plsc.VectorSubcoreMesh(core_axis_name="c", subcore_axis_name="s"),
        compiler_params=pltpu.CompilerParams(needs_layout_passes=False),
        scratch_types=[
            pltpu.VMEM((_D,), jnp.float32),
            pltpu.VMEM((_D,), jnp.float32),
            pltpu.VMEM((_D + 2 * _L,), jnp.float32),
            pltpu.VMEM((_NGRP * _L,), jnp.float32),
            pltpu.SemaphoreType.DMA,
            pltpu.SemaphoreType.DMA,
            pltpu.SemaphoreType.DMA,
            pltpu.SemaphoreType.DMA,
        ],
    )(x)


# speedup vs baseline: 1.0423x; 1.0423x over previous
import functools

import jax
import jax.numpy as jnp
from jax import lax
from jax.experimental import pallas as pl
from jax.experimental.pallas import tpu as pltpu
from jax.experimental.pallas import tpu_sc as plsc

_B = 128
_D = 32768
_L = 16
_GRP = 8
_NGRP = _D // (_L * _GRP)
_N_BISECT = 22
_WORKERS = 32
_ROWS_PER_W = _B // _WORKERS


def _process_row(row_v, cand_v, gmax_v):
    @plsc.parallel_loop(0, _NGRP, unroll=4, carry=row_v[pl.ds(0, _L)])
    def acc(g, acc):
        base = g * (_L * _GRP)
        c0 = jnp.maximum(row_v[pl.ds(base, _L)], row_v[pl.ds(base + _L, _L)])
        c1 = jnp.maximum(row_v[pl.ds(base + 2 * _L, _L)],
                         row_v[pl.ds(base + 3 * _L, _L)])
        c2 = jnp.maximum(row_v[pl.ds(base + 4 * _L, _L)],
                         row_v[pl.ds(base + 5 * _L, _L)])
        c3 = jnp.maximum(row_v[pl.ds(base + 6 * _L, _L)],
                         row_v[pl.ds(base + 7 * _L, _L)])
        gv = jnp.maximum(jnp.maximum(c0, c1), jnp.maximum(c2, c3))
        gmax_v[pl.ds(g * _L, _L)] = gv
        return jnp.maximum(acc, gv)

    m = jnp.max(acc)
    thr = m - 1.0

    @plsc.parallel_loop(0, _NGRP, unroll=2, carry=jnp.int32(0))
    def off(g, off):
        anyg = jnp.any(gmax_v[pl.ds(g * _L, _L)] > thr)

        def slow(off):
            base = g * (_L * _GRP)
            for c in range(_GRP):
                v = row_v[pl.ds(base + c * _L, _L)]
                mk = v > thr
                mi = jnp.where(mk, 1, 0).astype(jnp.int32)
                cs = plsc.cumsum(mi)
                pos = cs - 1 + off
                plsc.store_scatter(cand_v, [pos], v, mask=mk)
                off = off + jnp.sum(mi)
            return off

        return lax.cond(anyg, slow, lambda off: off, off)

    pad = jnp.full((_L,), 0.0, jnp.float32) + thr
    cand_v[pl.ds(off, _L)] = pad
    cand_v[pl.ds(off + _L, _L)] = pad
    nch = off // _L + 1

    def probe(t):
        def pb(i, a):
            return a + jnp.maximum(cand_v[pl.ds(i * _L, _L)] - t, 0.0)

        return jnp.sum(lax.fori_loop(0, nch, pb, jnp.zeros((_L,), jnp.float32)))

    def bis_body(_, carry):
        lo, hi = carry
        mid = 0.5 * (lo + hi)
        ge = probe(mid) >= 1.0
        lo = jnp.where(ge, mid, lo)
        hi = jnp.where(ge, hi, mid)
        return lo, hi

    lo, _hi = lax.fori_loop(0, _N_BISECT, bis_body, (thr, m))

    def ref_body(i, carry):
        ka, sa = carry
        v = cand_v[pl.ds(i * _L, _L)]
        mk = v > lo
        return (ka + jnp.where(mk, 1.0, 0.0), sa + jnp.where(mk, v, 0.0))

    ka, sa = lax.fori_loop(
        0, nch, ref_body,
        (jnp.zeros((_L,), jnp.float32), jnp.zeros((_L,), jnp.float32)))
    sa_b = jnp.zeros((_L,), jnp.float32) + jnp.sum(sa)
    ka_b = jnp.zeros((_L,), jnp.float32) + jnp.sum(ka)
    tau = (sa_b - 1.0) / ka_b

    @plsc.parallel_loop(0, _NGRP, unroll=4)
    def _(g):
        base = g * (_L * _GRP)
        for c in range(_GRP):
            idx = pl.ds(base + c * _L, _L)
            row_v[idx] = jnp.maximum(row_v[idx] - tau, 0.0)


def _sc_body(x_hbm, out_hbm, row_a, row_b, cand_v, gmax_v,
             in_a, in_b, out_a, out_b):
    wid = lax.axis_index("s") * 2 + lax.axis_index("c")
    base = wid * _ROWS_PER_W

    bufs = [(row_a, in_a, out_a), (row_b, in_b, out_b)]
    in_h = [None, None]
    out_h = [None, None]
    in_h[0] = pltpu.async_copy(x_hbm.at[base], row_a, in_a)
    for j in range(_ROWS_PER_W):
        buf, insem, outsem = bufs[j % 2]
        in_h[j % 2].wait()
        if j + 1 < _ROWS_PER_W:
            nbuf, ninsem, _ = bufs[(j + 1) % 2]
            if j >= 1:
                out_h[(j + 1) % 2].wait()
            in_h[(j + 1) % 2] = pltpu.async_copy(
                x_hbm.at[base + j + 1], nbuf, ninsem)
        _process_row(buf, cand_v, gmax_v)
        out_h[j % 2] = pltpu.async_copy(buf, out_hbm.at[base + j], outsem)
    out_h[(_ROWS_PER_W - 1) % 2].wait()
    out_h[(_ROWS_PER_W - 2) % 2].wait()


def kernel(x):
    return pl.kernel(
        _sc_body,
        out_type=jax.ShapeDtypeStruct((_B, _D), jnp.float32),
        mesh=plsc.VectorSubcoreMesh(core_axis_name="c", subcore_axis_name="s"),
        compiler_params=pltpu.CompilerParams(needs_layout_passes=False),
        scratch_types=[
            pltpu.VMEM((_D,), jnp.float32),
            pltpu.VMEM((_D,), jnp.float32),
            pltpu.VMEM((_D + 2 * _L,), jnp.float32),
            pltpu.VMEM((_NGRP * _L,), jnp.float32),
            pltpu.SemaphoreType.DMA,
            pltpu.SemaphoreType.DMA,
            pltpu.SemaphoreType.DMA,
            pltpu.SemaphoreType.DMA,
        ],
    )(x)


# speedup vs baseline: 1.2205x; 1.1709x over previous
import functools

import jax
import jax.numpy as jnp
from jax import lax
from jax.experimental import pallas as pl
from jax.experimental.pallas import tpu as pltpu
from jax.experimental.pallas import tpu_sc as plsc

_B = 128
_D = 32768
_L = 16
_GRP = 8
_NGRP = _D // (_L * _GRP)
_N_BISECT_SC = 22
_WORKERS = 32
_B_SC = 32
_ROWS_PER_W = _B_SC // _WORKERS
_N_BISECT_TC = 14
_TC_ROWS_PER_BLOCK = 48


def _process_row(row_v, cand_v, gmax_v):
    @plsc.parallel_loop(0, _NGRP, unroll=4, carry=row_v[pl.ds(0, _L)])
    def acc(g, acc):
        base = g * (_L * _GRP)
        c0 = jnp.maximum(row_v[pl.ds(base, _L)], row_v[pl.ds(base + _L, _L)])
        c1 = jnp.maximum(row_v[pl.ds(base + 2 * _L, _L)],
                         row_v[pl.ds(base + 3 * _L, _L)])
        c2 = jnp.maximum(row_v[pl.ds(base + 4 * _L, _L)],
                         row_v[pl.ds(base + 5 * _L, _L)])
        c3 = jnp.maximum(row_v[pl.ds(base + 6 * _L, _L)],
                         row_v[pl.ds(base + 7 * _L, _L)])
        gv = jnp.maximum(jnp.maximum(c0, c1), jnp.maximum(c2, c3))
        gmax_v[pl.ds(g * _L, _L)] = gv
        return jnp.maximum(acc, gv)

    m = jnp.max(acc)
    thr = m - 1.0

    @plsc.parallel_loop(0, _NGRP, unroll=2, carry=jnp.int32(0))
    def off(g, off):
        anyg = jnp.any(gmax_v[pl.ds(g * _L, _L)] > thr)

        def slow(off):
            base = g * (_L * _GRP)
            for c in range(_GRP):
                v = row_v[pl.ds(base + c * _L, _L)]
                mk = v > thr
                mi = jnp.where(mk, 1, 0).astype(jnp.int32)
                cs = plsc.cumsum(mi)
                pos = cs - 1 + off
                plsc.store_scatter(cand_v, [pos], v, mask=mk)
                off = off + jnp.sum(mi)
            return off

        return lax.cond(anyg, slow, lambda off: off, off)

    pad = jnp.full((_L,), 0.0, jnp.float32) + thr
    cand_v[pl.ds(off, _L)] = pad
    cand_v[pl.ds(off + _L, _L)] = pad
    nch = off // _L + 1

    def probe(t):
        def pb(i, a):
            return a + jnp.maximum(cand_v[pl.ds(i * _L, _L)] - t, 0.0)

        return jnp.sum(lax.fori_loop(0, nch, pb, jnp.zeros((_L,), jnp.float32)))

    def bis_body(_, carry):
        lo, hi = carry
        mid = 0.5 * (lo + hi)
        ge = probe(mid) >= 1.0
        lo = jnp.where(ge, mid, lo)
        hi = jnp.where(ge, hi, mid)
        return lo, hi

    lo, _hi = lax.fori_loop(0, _N_BISECT_SC, bis_body, (thr, m))

    def ref_body(i, carry):
        ka, sa = carry
        v = cand_v[pl.ds(i * _L, _L)]
        mk = v > lo
        return (ka + jnp.where(mk, 1.0, 0.0), sa + jnp.where(mk, v, 0.0))

    ka, sa = lax.fori_loop(
        0, nch, ref_body,
        (jnp.zeros((_L,), jnp.float32), jnp.zeros((_L,), jnp.float32)))
    sa_b = jnp.zeros((_L,), jnp.float32) + jnp.sum(sa)
    ka_b = jnp.zeros((_L,), jnp.float32) + jnp.sum(ka)
    tau = (sa_b - 1.0) / ka_b

    @plsc.parallel_loop(0, _NGRP, unroll=4)
    def _(g):
        base = g * (_L * _GRP)
        for c in range(_GRP):
            idx = pl.ds(base + c * _L, _L)
            row_v[idx] = jnp.maximum(row_v[idx] - tau, 0.0)


def _sc_body(x_hbm, out_hbm, row_a, row_b, cand_v, gmax_v,
             in_a, in_b, out_a, out_b):
    wid = lax.axis_index("s") * 2 + lax.axis_index("c")
    base = wid * _ROWS_PER_W

    bufs = [(row_a, in_a, out_a), (row_b, in_b, out_b)]
    in_h = [None, None]
    out_h = [None, None]
    in_h[0] = pltpu.async_copy(x_hbm.at[base], row_a, in_a)
    for j in range(_ROWS_PER_W):
        buf, insem, outsem = bufs[j % 2]
        in_h[j % 2].wait()
        if j + 1 < _ROWS_PER_W:
            nbuf, ninsem, _ = bufs[(j + 1) % 2]
            if j >= 1:
                out_h[(j + 1) % 2].wait()
            in_h[(j + 1) % 2] = pltpu.async_copy(
                x_hbm.at[base + j + 1], nbuf, ninsem)
        _process_row(buf, cand_v, gmax_v)
        out_h[j % 2] = pltpu.async_copy(buf, out_hbm.at[base + j], outsem)
    out_h[(_ROWS_PER_W - 1) % 2].wait()
    if _ROWS_PER_W >= 2:
        out_h[(_ROWS_PER_W - 2) % 2].wait()


def _sc_kernel(x):
    return pl.kernel(
        _sc_body,
        out_type=jax.ShapeDtypeStruct((_B_SC, _D), jnp.float32),
        mesh=plsc.VectorSubcoreMesh(core_axis_name="c", subcore_axis_name="s"),
        compiler_params=pltpu.CompilerParams(needs_layout_passes=False),
        scratch_types=[
            pltpu.VMEM((_D,), jnp.float32),
            pltpu.VMEM((_D,), jnp.float32),
            pltpu.VMEM((_D + 2 * _L,), jnp.float32),
            pltpu.VMEM((_NGRP * _L,), jnp.float32),
            pltpu.SemaphoreType.DMA,
            pltpu.SemaphoreType.DMA,
            pltpu.SemaphoreType.DMA,
            pltpu.SemaphoreType.DMA,
        ],
    )(x)


def _tc_block(x_ref, o_ref):
    xb = x_ref[...]
    m = jnp.max(xb, axis=-1, keepdims=True)
    lo = m - 1.0
    hi = m

    def body(_, carry):
        lo, hi = carry
        mid = 0.5 * (lo + hi)
        f = jnp.sum(jnp.maximum(xb - mid, 0.0), axis=-1, keepdims=True)
        ge = f >= 1.0
        return jnp.where(ge, mid, lo), jnp.where(ge, hi, mid)

    lo, hi = jax.lax.fori_loop(0, _N_BISECT_TC, body, (lo, hi))
    mask = xb > lo
    k = jnp.sum(mask.astype(jnp.float32), axis=-1, keepdims=True)
    s = jnp.sum(jnp.where(mask, xb, 0.0), axis=-1, keepdims=True)
    tau = (s - 1.0) / k
    o_ref[...] = jnp.maximum(xb - tau, 0.0)


def _tc_kernel(x):
    b, d = x.shape
    rows = _TC_ROWS_PER_BLOCK
    return pl.pallas_call(
        _tc_block,
        grid=(b // rows,),
        in_specs=[pl.BlockSpec((rows, d), lambda i: (i, 0))],
        out_specs=pl.BlockSpec((rows, d), lambda i: (i, 0)),
        out_shape=jax.ShapeDtypeStruct((b, d), x.dtype),
    )(x)


def kernel(x):
    sc_out = _sc_kernel(x[:_B_SC])
    tc_out = _tc_kernel(x[_B_SC:])
    return jnp.concatenate([sc_out, tc_out], axis=0)


# speedup vs baseline: 1.4148x; 1.1593x over previous
import functools

import jax
import jax.numpy as jnp
from jax import lax
from jax.experimental import pallas as pl
from jax.experimental.pallas import tpu as pltpu
from jax.experimental.pallas import tpu_sc as plsc

_B = 128
_D = 32768
_L = 16
_GRP = 8
_NGRP = _D // (_L * _GRP)
_N_BISECT_SC = 22
_WORKERS = 32
_B_SC = 32
_ROWS_PER_W = _B_SC // _WORKERS
_N_BISECT_TC = 14
_TC_ROWS_PER_BLOCK = 32


def _process_row(row_v, cand_v, gmax_v):
    @plsc.parallel_loop(0, _NGRP, unroll=4, carry=row_v[pl.ds(0, _L)])
    def acc(g, acc):
        base = g * (_L * _GRP)
        c0 = jnp.maximum(row_v[pl.ds(base, _L)], row_v[pl.ds(base + _L, _L)])
        c1 = jnp.maximum(row_v[pl.ds(base + 2 * _L, _L)],
                         row_v[pl.ds(base + 3 * _L, _L)])
        c2 = jnp.maximum(row_v[pl.ds(base + 4 * _L, _L)],
                         row_v[pl.ds(base + 5 * _L, _L)])
        c3 = jnp.maximum(row_v[pl.ds(base + 6 * _L, _L)],
                         row_v[pl.ds(base + 7 * _L, _L)])
        gv = jnp.maximum(jnp.maximum(c0, c1), jnp.maximum(c2, c3))
        gmax_v[pl.ds(g * _L, _L)] = gv
        return jnp.maximum(acc, gv)

    m = jnp.max(acc)
    thr = m - 1.0

    @plsc.parallel_loop(0, _NGRP, unroll=2, carry=jnp.int32(0))
    def off(g, off):
        anyg = jnp.any(gmax_v[pl.ds(g * _L, _L)] > thr)

        def slow(off):
            base = g * (_L * _GRP)
            for c in range(_GRP):
                v = row_v[pl.ds(base + c * _L, _L)]
                mk = v > thr
                mi = jnp.where(mk, 1, 0).astype(jnp.int32)
                cs = plsc.cumsum(mi)
                pos = cs - 1 + off
                plsc.store_scatter(cand_v, [pos], v, mask=mk)
                off = off + jnp.sum(mi)
            return off

        return lax.cond(anyg, slow, lambda off: off, off)

    pad = jnp.full((_L,), 0.0, jnp.float32) + thr
    cand_v[pl.ds(off, _L)] = pad
    cand_v[pl.ds(off + _L, _L)] = pad
    nch = off // _L + 1

    def probe(t):
        def pb(i, a):
            return a + jnp.maximum(cand_v[pl.ds(i * _L, _L)] - t, 0.0)

        return jnp.sum(lax.fori_loop(0, nch, pb, jnp.zeros((_L,), jnp.float32)))

    def bis_body(_, carry):
        lo, hi = carry
        mid = 0.5 * (lo + hi)
        ge = probe(mid) >= 1.0
        lo = jnp.where(ge, mid, lo)
        hi = jnp.where(ge, hi, mid)
        return lo, hi

    lo, _hi = lax.fori_loop(0, _N_BISECT_SC, bis_body, (thr, m))

    def ref_body(i, carry):
        ka, sa = carry
        v = cand_v[pl.ds(i * _L, _L)]
        mk = v > lo
        return (ka + jnp.where(mk, 1.0, 0.0), sa + jnp.where(mk, v, 0.0))

    ka, sa = lax.fori_loop(
        0, nch, ref_body,
        (jnp.zeros((_L,), jnp.float32), jnp.zeros((_L,), jnp.float32)))
    sa_b = jnp.zeros((_L,), jnp.float32) + jnp.sum(sa)
    ka_b = jnp.zeros((_L,), jnp.float32) + jnp.sum(ka)
    tau = (sa_b - 1.0) / ka_b

    @plsc.parallel_loop(0, _NGRP, unroll=4)
    def _(g):
        base = g * (_L * _GRP)
        for c in range(_GRP):
            idx = pl.ds(base + c * _L, _L)
            row_v[idx] = jnp.maximum(row_v[idx] - tau, 0.0)


def _sc_body(x_hbm, out_hbm, row_a, row_b, cand_v, gmax_v,
             in_a, in_b, out_a, out_b):
    wid = lax.axis_index("s") * 2 + lax.axis_index("c")
    base = wid * _ROWS_PER_W

    bufs = [(row_a, in_a, out_a), (row_b, in_b, out_b)]
    in_h = [None, None]
    out_h = [None, None]
    in_h[0] = pltpu.async_copy(x_hbm.at[base], row_a, in_a)
    for j in range(_ROWS_PER_W):
        buf, insem, outsem = bufs[j % 2]
        in_h[j % 2].wait()
        if j + 1 < _ROWS_PER_W:
            nbuf, ninsem, _ = bufs[(j + 1) % 2]
            if j >= 1:
                out_h[(j + 1) % 2].wait()
            in_h[(j + 1) % 2] = pltpu.async_copy(
                x_hbm.at[base + j + 1], nbuf, ninsem)
        _process_row(buf, cand_v, gmax_v)
        out_h[j % 2] = pltpu.async_copy(buf, out_hbm.at[base + j], outsem)
    out_h[(_ROWS_PER_W - 1) % 2].wait()
    if _ROWS_PER_W >= 2:
        out_h[(_ROWS_PER_W - 2) % 2].wait()


def _sc_kernel(x):
    return pl.kernel(
        _sc_body,
        out_type=jax.ShapeDtypeStruct((_B_SC, _D), jnp.float32),
        mesh=plsc.VectorSubcoreMesh(core_axis_name="c", subcore_axis_name="s"),
        compiler_params=pltpu.CompilerParams(needs_layout_passes=False),
        scratch_types=[
            pltpu.VMEM((_D,), jnp.float32),
            pltpu.VMEM((_D,), jnp.float32),
            pltpu.VMEM((_D + 2 * _L,), jnp.float32),
            pltpu.VMEM((_NGRP * _L,), jnp.float32),
            pltpu.SemaphoreType.DMA,
            pltpu.SemaphoreType.DMA,
            pltpu.SemaphoreType.DMA,
            pltpu.SemaphoreType.DMA,
        ],
    )(x)


def _tc_block(x_ref, o_ref):
    xb = x_ref[...]
    m = jnp.max(xb, axis=-1, keepdims=True)
    lo = m - 1.0
    hi = m

    def body(_, carry):
        lo, hi = carry
        mid = 0.5 * (lo + hi)
        f = jnp.sum(jnp.maximum(xb - mid, 0.0), axis=-1, keepdims=True)
        ge = f >= 1.0
        return jnp.where(ge, mid, lo), jnp.where(ge, hi, mid)

    lo, hi = jax.lax.fori_loop(0, _N_BISECT_TC, body, (lo, hi))
    mask = xb > lo
    k = jnp.sum(mask.astype(jnp.float32), axis=-1, keepdims=True)
    s = jnp.sum(jnp.where(mask, xb, 0.0), axis=-1, keepdims=True)
    tau = (s - 1.0) / k
    o_ref[...] = jnp.maximum(xb - tau, 0.0)


def _tc_kernel(x):
    b, d = x.shape
    rows = _TC_ROWS_PER_BLOCK
    skip = _B_SC // rows
    return pl.pallas_call(
        _tc_block,
        grid=(b // rows - skip,),
        in_specs=[pl.BlockSpec((rows, d), lambda i: (i + skip, 0))],
        out_specs=pl.BlockSpec((rows, d), lambda i: (i, 0)),
        out_shape=jax.ShapeDtypeStruct((b - _B_SC, d), x.dtype),
    )(x)


def kernel(x):
    sc_out = _sc_kernel(x)
    tc_out = _tc_kernel(x)
    return jnp.concatenate([sc_out, tc_out], axis=0)


# speedup vs baseline: 1.5179x; 1.0729x over previous
import functools

import jax
import jax.numpy as jnp
from jax import lax
from jax.experimental import pallas as pl
from jax.experimental.pallas import tpu as pltpu
from jax.experimental.pallas import tpu_sc as plsc

_B = 128
_D = 32768
_L = 16
_GRP = 8
_NGRP = _D // (_L * _GRP)
_N_BISECT_SC = 22
_WORKERS = 32
_B_SC = 32
_ROWS_PER_W = _B_SC // _WORKERS
_N_BISECT_TC = 14
_TC_ROWS_PER_BLOCK = 48


def _process_row(row_v, cand_v, gmax_v):
    @plsc.parallel_loop(0, _NGRP, unroll=4, carry=row_v[pl.ds(0, _L)])
    def acc(g, acc):
        base = g * (_L * _GRP)
        c0 = jnp.maximum(row_v[pl.ds(base, _L)], row_v[pl.ds(base + _L, _L)])
        c1 = jnp.maximum(row_v[pl.ds(base + 2 * _L, _L)],
                         row_v[pl.ds(base + 3 * _L, _L)])
        c2 = jnp.maximum(row_v[pl.ds(base + 4 * _L, _L)],
                         row_v[pl.ds(base + 5 * _L, _L)])
        c3 = jnp.maximum(row_v[pl.ds(base + 6 * _L, _L)],
                         row_v[pl.ds(base + 7 * _L, _L)])
        gv = jnp.maximum(jnp.maximum(c0, c1), jnp.maximum(c2, c3))
        gmax_v[pl.ds(g * _L, _L)] = gv
        return jnp.maximum(acc, gv)

    m = jnp.max(acc)
    thr = m - 1.0

    @plsc.parallel_loop(0, _NGRP, unroll=2, carry=jnp.int32(0))
    def off(g, off):
        anyg = jnp.any(gmax_v[pl.ds(g * _L, _L)] > thr)

        def slow(off):
            base = g * (_L * _GRP)
            for c in range(_GRP):
                v = row_v[pl.ds(base + c * _L, _L)]
                mk = v > thr
                mi = jnp.where(mk, 1, 0).astype(jnp.int32)
                cs = plsc.cumsum(mi)
                pos = cs - 1 + off
                plsc.store_scatter(cand_v, [pos], v, mask=mk)
                off = off + jnp.sum(mi)
            return off

        return lax.cond(anyg, slow, lambda off: off, off)

    pad = jnp.full((_L,), 0.0, jnp.float32) + thr
    cand_v[pl.ds(off, _L)] = pad
    cand_v[pl.ds(off + _L, _L)] = pad
    nch = off // _L + 1

    def probe(t):
        def pb(i, a):
            return a + jnp.maximum(cand_v[pl.ds(i * _L, _L)] - t, 0.0)

        return jnp.sum(lax.fori_loop(0, nch, pb, jnp.zeros((_L,), jnp.float32)))

    def bis_body(_, carry):
        lo, hi = carry
        mid = 0.5 * (lo + hi)
        ge = probe(mid) >= 1.0
        lo = jnp.where(ge, mid, lo)
        hi = jnp.where(ge, hi, mid)
        return lo, hi

    lo, _hi = lax.fori_loop(0, _N_BISECT_SC, bis_body, (thr, m))

    def ref_body(i, carry):
        ka, sa = carry
        v = cand_v[pl.ds(i * _L, _L)]
        mk = v > lo
        return (ka + jnp.where(mk, 1.0, 0.0), sa + jnp.where(mk, v, 0.0))

    ka, sa = lax.fori_loop(
        0, nch, ref_body,
        (jnp.zeros((_L,), jnp.float32), jnp.zeros((_L,), jnp.float32)))
    sa_b = jnp.zeros((_L,), jnp.float32) + jnp.sum(sa)
    ka_b = jnp.zeros((_L,), jnp.float32) + jnp.sum(ka)
    tau = (sa_b - 1.0) / ka_b

    @plsc.parallel_loop(0, _NGRP, unroll=4)
    def _(g):
        base = g * (_L * _GRP)
        for c in range(_GRP):
            idx = pl.ds(base + c * _L, _L)
            row_v[idx] = jnp.maximum(row_v[idx] - tau, 0.0)


def _sc_body(x_hbm, out_hbm, row_a, row_b, cand_v, gmax_v,
             in_a, in_b, out_a, out_b):
    wid = lax.axis_index("s") * 2 + lax.axis_index("c")
    base = wid * _ROWS_PER_W
    in_base = (_B - _B_SC) + base

    bufs = [(row_a, in_a, out_a), (row_b, in_b, out_b)]
    in_h = [None, None]
    out_h = [None, None]
    in_h[0] = pltpu.async_copy(x_hbm.at[in_base], row_a, in_a)
    for j in range(_ROWS_PER_W):
        buf, insem, outsem = bufs[j % 2]
        in_h[j % 2].wait()
        if j + 1 < _ROWS_PER_W:
            nbuf, ninsem, _ = bufs[(j + 1) % 2]
            if j >= 1:
                out_h[(j + 1) % 2].wait()
            in_h[(j + 1) % 2] = pltpu.async_copy(
                x_hbm.at[in_base + j + 1], nbuf, ninsem)
        _process_row(buf, cand_v, gmax_v)
        out_h[j % 2] = pltpu.async_copy(buf, out_hbm.at[base + j], outsem)
    out_h[(_ROWS_PER_W - 1) % 2].wait()
    if _ROWS_PER_W >= 2:
        out_h[(_ROWS_PER_W - 2) % 2].wait()


def _sc_kernel(x):
    return pl.kernel(
        _sc_body,
        out_type=jax.ShapeDtypeStruct((_B_SC, _D), jnp.float32),
        mesh=plsc.VectorSubcoreMesh(core_axis_name="c", subcore_axis_name="s"),
        compiler_params=pltpu.CompilerParams(needs_layout_passes=False),
        scratch_types=[
            pltpu.VMEM((_D,), jnp.float32),
            pltpu.VMEM((_D,), jnp.float32),
            pltpu.VMEM((_D + 2 * _L,), jnp.float32),
            pltpu.VMEM((_NGRP * _L,), jnp.float32),
            pltpu.SemaphoreType.DMA,
            pltpu.SemaphoreType.DMA,
            pltpu.SemaphoreType.DMA,
            pltpu.SemaphoreType.DMA,
        ],
    )(x)


def _tc_block(x_ref, o_ref):
    xb = x_ref[...]
    m = jnp.max(xb, axis=-1, keepdims=True)
    lo = m - 1.0
    hi = m

    def body(_, carry):
        lo, hi = carry
        mid = 0.5 * (lo + hi)
        f = jnp.sum(jnp.maximum(xb - mid, 0.0), axis=-1, keepdims=True)
        ge = f >= 1.0
        return jnp.where(ge, mid, lo), jnp.where(ge, hi, mid)

    lo, hi = jax.lax.fori_loop(0, _N_BISECT_TC, body, (lo, hi))
    mask = xb > lo
    k = jnp.sum(mask.astype(jnp.float32), axis=-1, keepdims=True)
    s = jnp.sum(jnp.where(mask, xb, 0.0), axis=-1, keepdims=True)
    tau = (s - 1.0) / k
    o_ref[...] = jnp.maximum(xb - tau, 0.0)


def _tc_kernel(x):
    b, d = x.shape
    rows = _TC_ROWS_PER_BLOCK
    return pl.pallas_call(
        _tc_block,
        grid=((b - _B_SC) // rows,),
        in_specs=[pl.BlockSpec((rows, d), lambda i: (i, 0))],
        out_specs=pl.BlockSpec((rows, d), lambda i: (i, 0)),
        out_shape=jax.ShapeDtypeStruct((b - _B_SC, d), x.dtype),
    )(x)


def kernel(x):
    sc_out = _sc_kernel(x)
    tc_out = _tc_kernel(x)
    return jnp.concatenate([tc_out, sc_out], axis=0)
